# baseline (speedup 1.0000x reference)
import functools

import jax
import jax.numpy as jnp
from jax import lax
from jax.experimental import pallas as pl
from jax.experimental.pallas import tpu as pltpu
from jax.experimental.pallas import tpu_sc as plsc

NUM_CHANNELS = 96
NUM_PIXELS = 512 * 512
NUM_SAMPLES = 26214

NW = 32
BPW = 832
B_PAD = NW * BPW
CHUNK = 104
NCHUNK = BPW // CHUNK


def _sc_gather(img_flat_hbm, idxp_hbm):
    mesh = plsc.VectorSubcoreMesh(core_axis_name="c", subcore_axis_name="s")

    @functools.partial(
        pl.kernel,
        mesh=mesh,
        out_type=jax.ShapeDtypeStruct((NUM_CHANNELS * B_PAD,), jnp.float32),
        scratch_types=[
            pltpu.VMEM((BPW,), jnp.int32),
            pltpu.VMEM((BPW,), jnp.int32),
            pltpu.VMEM((BPW,), jnp.float32),
            pltpu.SemaphoreType.DMA,
        ],
    )
    def k(img_hbm, idx_hbm, out_hbm, idx_v, fi_v, row_v, sem):
        cid = lax.axis_index("c")
        sid = lax.axis_index("s")
        wid = sid * 2 + cid
        base = wid * BPW
        pltpu.sync_copy(idx_hbm.at[pl.ds(base, BPW)], idx_v)

        def chan(c, carry):
            off = c * NUM_PIXELS
            for i in range(BPW // 16):
                sl = pl.ds(i * 16, 16)
                fi_v[sl] = idx_v[sl] + off
            copies = []
            for j in range(NCHUNK):
                sl = pl.ds(j * CHUNK, CHUNK)
                copies.append(
                    pltpu.async_copy(img_hbm.at[fi_v.at[sl]], row_v.at[sl], sem)
                )
            for cp in copies:
                cp.wait()
            dst = pl.multiple_of(c * B_PAD + base, 8)
            pltpu.sync_copy(row_v, out_hbm.at[pl.ds(dst, BPW)])
            return carry

        lax.fori_loop(0, NUM_CHANNELS, chan, 0)

    return k(img_flat_hbm, idxp_hbm)


def kernel(image_flat, idx):
    idx_i = idx.astype(jnp.int32)
    idxp = jnp.pad(idx_i, (0, B_PAD - NUM_SAMPLES))
    out_flat = _sc_gather(image_flat.reshape(-1), idxp)
    image_subset = out_flat.reshape(NUM_CHANNELS, B_PAD)[:, :NUM_SAMPLES]
    return (image_subset, idx)


# speedup vs baseline: 1.4433x; 1.4433x over previous
import functools

import jax
import jax.numpy as jnp
from jax import lax
from jax.experimental import pallas as pl
from jax.experimental.pallas import tpu as pltpu
from jax.experimental.pallas import tpu_sc as plsc

NUM_CHANNELS = 96
NUM_PIXELS = 512 * 512
NUM_SAMPLES = 26214

NW = 32
BPW = 832
B_PAD = NW * BPW
CHUNK = 104
NCHUNK = BPW // CHUNK


def _sc_gather(img_flat_hbm, idxp_hbm):
    mesh = plsc.VectorSubcoreMesh(core_axis_name="c", subcore_axis_name="s")

    @functools.partial(
        pl.kernel,
        mesh=mesh,
        out_type=jax.ShapeDtypeStruct((NUM_CHANNELS * B_PAD,), jnp.float32),
        scratch_types=[
            pltpu.VMEM((BPW,), jnp.int32),
            pltpu.VMEM((NUM_CHANNELS * BPW,), jnp.float32),
            pltpu.SemaphoreType.DMA,
            pltpu.SemaphoreType.DMA,
        ],
    )
    def k(img_hbm, idx_hbm, out_hbm, idx_v, buf_v, sem_g, sem_w):
        cid = lax.axis_index("c")
        sid = lax.axis_index("s")
        wid = sid * 2 + cid
        base = wid * BPW
        pltpu.sync_copy(idx_hbm.at[pl.ds(base, BPW)], idx_v)

        def fire(c, carry):
            src = img_hbm.at[pl.ds(pl.multiple_of(c * NUM_PIXELS, 8),
                                   NUM_PIXELS)]
            for j in range(NCHUNK):
                sl = pl.ds(j * CHUNK, CHUNK)
                dst = pl.ds(c * BPW + j * CHUNK, CHUNK)
                pltpu.async_copy(src.at[idx_v.at[sl]], buf_v.at[dst], sem_g)
            return carry

        lax.fori_loop(0, NUM_CHANNELS, fire, 0)
        pltpu.make_async_copy(out_hbm.at[pl.ds(0, NUM_CHANNELS * BPW)],
                              buf_v, sem_g).wait()

        def write(c, carry):
            dst = pl.ds(pl.multiple_of(c * B_PAD + base, 8), BPW)
            pltpu.async_copy(buf_v.at[pl.ds(c * BPW, BPW)],
                             out_hbm.at[dst], sem_w)
            return carry

        lax.fori_loop(0, NUM_CHANNELS, write, 0)
        pltpu.make_async_copy(out_hbm.at[pl.ds(0, NUM_CHANNELS * BPW)],
                              buf_v, sem_w).wait()

    return k(img_flat_hbm, idxp_hbm)


def kernel(image_flat, idx):
    idx_i = idx.astype(jnp.int32)
    idxp = jnp.pad(idx_i, (0, B_PAD - NUM_SAMPLES))
    out_flat = _sc_gather(image_flat.reshape(-1), idxp)
    image_subset = out_flat.reshape(NUM_CHANNELS, B_PAD)[:, :NUM_SAMPLES]
    return (image_subset, idx)


# speedup vs baseline: 1.4441x; 1.0005x over previous
import functools

import jax
import jax.numpy as jnp
from jax import lax
from jax.experimental import pallas as pl
from jax.experimental.pallas import tpu as pltpu
from jax.experimental.pallas import tpu_sc as plsc

NUM_CHANNELS = 96
NUM_PIXELS = 512 * 512
NUM_SAMPLES = 26214

NW = 32
BPW = 832
B_PAD = NW * BPW
CHUNK = 832
NCHUNK = BPW // CHUNK


def _sc_gather(img_flat_hbm, idxp_hbm):
    mesh = plsc.VectorSubcoreMesh(core_axis_name="c", subcore_axis_name="s")

    @functools.partial(
        pl.kernel,
        mesh=mesh,
        out_type=jax.ShapeDtypeStruct((NUM_CHANNELS * B_PAD,), jnp.float32),
        scratch_types=[
            pltpu.VMEM((BPW,), jnp.int32),
            pltpu.VMEM((NUM_CHANNELS * BPW,), jnp.float32),
            pltpu.SemaphoreType.DMA,
            pltpu.SemaphoreType.DMA,
        ],
    )
    def k(img_hbm, idx_hbm, out_hbm, idx_v, buf_v, sem_g, sem_w):
        cid = lax.axis_index("c")
        sid = lax.axis_index("s")
        wid = sid * 2 + cid
        base = wid * BPW
        pltpu.sync_copy(idx_hbm.at[pl.ds(base, BPW)], idx_v)

        def fire(c, carry):
            src = img_hbm.at[pl.ds(pl.multiple_of(c * NUM_PIXELS, 8),
                                   NUM_PIXELS)]
            for j in range(NCHUNK):
                sl = pl.ds(j * CHUNK, CHUNK)
                dst = pl.ds(c * BPW + j * CHUNK, CHUNK)
                pltpu.async_copy(src.at[idx_v.at[sl]], buf_v.at[dst], sem_g)
            return carry

        lax.fori_loop(0, NUM_CHANNELS, fire, 0)
        pltpu.make_async_copy(out_hbm.at[pl.ds(0, NUM_CHANNELS * BPW)],
                              buf_v, sem_g).wait()

        def write(c, carry):
            dst = pl.ds(pl.multiple_of(c * B_PAD + base, 8), BPW)
            pltpu.async_copy(buf_v.at[pl.ds(c * BPW, BPW)],
                             out_hbm.at[dst], sem_w)
            return carry

        lax.fori_loop(0, NUM_CHANNELS, write, 0)
        pltpu.make_async_copy(out_hbm.at[pl.ds(0, NUM_CHANNELS * BPW)],
                              buf_v, sem_w).wait()

    return k(img_flat_hbm, idxp_hbm)


def kernel(image_flat, idx):
    idx_i = idx.astype(jnp.int32)
    idxp = jnp.pad(idx_i, (0, B_PAD - NUM_SAMPLES))
    out_flat = _sc_gather(image_flat.reshape(-1), idxp)
    image_subset = out_flat.reshape(NUM_CHANNELS, B_PAD)[:, :NUM_SAMPLES]
    return (image_subset, idx)


# speedup vs baseline: 2.0533x; 1.4219x over previous
import functools

import jax
import jax.numpy as jnp
from jax import lax
from jax.experimental import pallas as pl
from jax.experimental.pallas import tpu as pltpu
from jax.experimental.pallas import tpu_sc as plsc

NUM_CHANNELS = 96
NUM_PIXELS = 512 * 512
NUM_SAMPLES = 26214

NC = 2
NS = 16
CPC = NUM_CHANNELS // NC
TPW = 1640
S_PAD = NS * TPW
SEG = NUM_PIXELS // NS


def _sc_gather(img_flat_hbm, idxp_hbm):
    mesh = plsc.VectorSubcoreMesh(core_axis_name="c", subcore_axis_name="s")

    @functools.partial(
        pl.kernel,
        mesh=mesh,
        out_type=jax.ShapeDtypeStruct((NUM_CHANNELS * S_PAD,), jnp.float32),
        scratch_types=[
            pltpu.VMEM((TPW,), jnp.int32),
            pltpu.VMEM((CPC * TPW,), jnp.float32),
            pltpu.VMEM_SHARED((2 * NUM_PIXELS,), jnp.float32),
            pltpu.SemaphoreType.DMA,
            pltpu.SemaphoreType.DMA,
            pltpu.SemaphoreType.DMA,
        ],
    )
    def k(img_hbm, idx_hbm, out_hbm, idx_v, res_v, rows_s, sem_s, sem_g, sem_w):
        cid = lax.axis_index("c")
        sid = lax.axis_index("s")
        c0 = cid * CPC
        pltpu.sync_copy(idx_hbm.at[pl.ds(sid * TPW, TPW)], idx_v)

        def stage(k_next, slot):
            src = pl.ds(pl.multiple_of((c0 + k_next) * NUM_PIXELS + sid * SEG, 8),
                        SEG)
            dst = pl.ds(pl.multiple_of(slot * NUM_PIXELS + sid * SEG, 8), SEG)
            pltpu.async_copy(img_hbm.at[src], rows_s.at[dst], sem_s)

        stage(0, 0)
        pltpu.make_async_copy(img_hbm.at[pl.ds(0, SEG)],
                              rows_s.at[pl.ds(0, SEG)], sem_s).wait()
        plsc.subcore_barrier()

        def chan(kk, carry):
            slot = lax.rem(kk, 2)

            @pl.when(kk < CPC - 1)
            def _():
                stage(kk + 1, 1 - slot)

            src = rows_s.at[pl.ds(pl.multiple_of(slot * NUM_PIXELS, 8),
                                  NUM_PIXELS)]
            dst = res_v.at[pl.ds(kk * TPW, TPW)]
            pltpu.async_copy(src.at[idx_v], dst, sem_g).wait()

            @pl.when(kk < CPC - 1)
            def _():
                pltpu.make_async_copy(img_hbm.at[pl.ds(0, SEG)],
                                      rows_s.at[pl.ds(0, SEG)], sem_s).wait()

            plsc.subcore_barrier()
            return carry

        lax.fori_loop(0, CPC, chan, 0)

        def write(kk, carry):
            dst = pl.ds(pl.multiple_of((c0 + kk) * S_PAD + sid * TPW, 8), TPW)
            pltpu.async_copy(res_v.at[pl.ds(kk * TPW, TPW)],
                             out_hbm.at[dst], sem_w)
            return carry

        lax.fori_loop(0, CPC, write, 0)
        pltpu.make_async_copy(img_hbm.at[pl.ds(0, CPC * TPW)],
                              res_v, sem_w).wait()

    return k(img_flat_hbm, idxp_hbm)


def kernel(image_flat, idx):
    idx_i = idx.astype(jnp.int32)
    idxp = jnp.pad(idx_i, (0, S_PAD - NUM_SAMPLES))
    out_flat = _sc_gather(image_flat.reshape(-1), idxp)
    image_subset = out_flat.reshape(NUM_CHANNELS, S_PAD)[:, :NUM_SAMPLES]
    return (image_subset, idx)


# speedup vs baseline: 2.4828x; 1.2092x over previous
import functools

import jax
import jax.numpy as jnp
from jax import lax
from jax.experimental import pallas as pl
from jax.experimental.pallas import tpu as pltpu
from jax.experimental.pallas import tpu_sc as plsc

NUM_CHANNELS = 96
NUM_PIXELS = 512 * 512
NUM_SAMPLES = 26214

NC = 2
NS = 16
CPC = NUM_CHANNELS // NC
TPW = 1640
S_PAD = NS * TPW
SEG = NUM_PIXELS // NS
NSLOT = 3


def _sc_gather(img_flat_hbm, idxp_hbm):
    mesh = plsc.VectorSubcoreMesh(core_axis_name="c", subcore_axis_name="s")

    @functools.partial(
        pl.kernel,
        mesh=mesh,
        out_type=jax.ShapeDtypeStruct((NUM_CHANNELS * S_PAD,), jnp.float32),
        scratch_types=[
            pltpu.VMEM((TPW,), jnp.int32),
            pltpu.VMEM((CPC * TPW,), jnp.float32),
            pltpu.VMEM_SHARED((NSLOT * NUM_PIXELS,), jnp.float32),
            pltpu.SemaphoreType.DMA,
            pltpu.SemaphoreType.DMA,
            pltpu.SemaphoreType.DMA,
            pltpu.SemaphoreType.DMA,
            pltpu.SemaphoreType.DMA,
        ],
    )
    def k(img_hbm, idx_hbm, out_hbm, idx_v, res_v, rows_s,
          sem_s0, sem_s1, sem_g0, sem_g1, sem_w):
        cid = lax.axis_index("c")
        sid = lax.axis_index("s")
        c0 = cid * CPC
        pltpu.sync_copy(idx_hbm.at[pl.ds(sid * TPW, TPW)], idx_v)

        def stage(k_next, sem):
            slot = lax.rem(k_next, NSLOT)
            src = pl.ds(pl.multiple_of((c0 + k_next) * NUM_PIXELS + sid * SEG, 8),
                        SEG)
            dst = pl.ds(pl.multiple_of(slot * NUM_PIXELS + sid * SEG, 8), SEG)
            pltpu.async_copy(img_hbm.at[src], rows_s.at[dst], sem)

        def wait_stage(sem):
            pltpu.make_async_copy(img_hbm.at[pl.ds(0, SEG)],
                                  rows_s.at[pl.ds(0, SEG)], sem).wait()

        def fire_gather(kk, sem):
            slot = lax.rem(kk, NSLOT)
            src = rows_s.at[pl.ds(pl.multiple_of(slot * NUM_PIXELS, 8),
                                  NUM_PIXELS)]
            dst = res_v.at[pl.ds(kk * TPW, TPW)]
            pltpu.async_copy(src.at[idx_v], dst, sem)

        def wait_gather(sem):
            pltpu.make_async_copy(img_hbm.at[pl.ds(0, TPW)],
                                  res_v.at[pl.ds(0, TPW)], sem).wait()

        def fire_write(kk):
            dst = pl.ds(pl.multiple_of((c0 + kk) * S_PAD + sid * TPW, 8), TPW)
            pltpu.async_copy(res_v.at[pl.ds(kk * TPW, TPW)],
                             out_hbm.at[dst], sem_w)

        stage(0, sem_s0)
        stage(1, sem_s1)

        def chan(kk, carry):
            even = lax.rem(kk, 2) == 0

            @pl.when(even)
            def _():
                wait_stage(sem_s0)

            @pl.when(jnp.logical_not(even))
            def _():
                wait_stage(sem_s1)

            @pl.when(jnp.logical_and(kk >= 1, jnp.logical_not(even)))
            def _():
                wait_gather(sem_g0)

            @pl.when(jnp.logical_and(kk >= 1, even))
            def _():
                wait_gather(sem_g1)

            @pl.when(kk >= 1)
            def _():
                fire_write(kk - 1)

            plsc.subcore_barrier()

            @pl.when(even)
            def _():
                fire_gather(kk, sem_g0)

                @pl.when(kk < CPC - 2)
                def _():
                    stage(kk + 2, sem_s0)

            @pl.when(jnp.logical_not(even))
            def _():
                fire_gather(kk, sem_g1)

                @pl.when(kk < CPC - 2)
                def _():
                    stage(kk + 2, sem_s1)

            return carry

        lax.fori_loop(0, CPC, chan, 0)

        wait_gather(sem_g1)
        fire_write(CPC - 1)
        pltpu.make_async_copy(img_hbm.at[pl.ds(0, CPC * TPW)],
                              res_v, sem_w).wait()

    return k(img_flat_hbm, idxp_hbm)


def kernel(image_flat, idx):
    idx_i = idx.astype(jnp.int32)
    idxp = jnp.pad(idx_i, (0, S_PAD - NUM_SAMPLES))
    out_flat = _sc_gather(image_flat.reshape(-1), idxp)
    image_subset = out_flat.reshape(NUM_CHANNELS, S_PAD)[:, :NUM_SAMPLES]
    return (image_subset, idx)


# speedup vs baseline: 2.4858x; 1.0012x over previous
import functools

import jax
import jax.numpy as jnp
from jax import lax
from jax.experimental import pallas as pl
from jax.experimental.pallas import tpu as pltpu
from jax.experimental.pallas import tpu_sc as plsc

NUM_CHANNELS = 96
NUM_PIXELS = 512 * 512
NUM_SAMPLES = 26214

NC = 2
NS = 16
CPC = NUM_CHANNELS // NC
TPW = 1640
S_PAD = NS * TPW
SEG = NUM_PIXELS // NS
NSLOT = 3


def _sc_gather(img_flat_hbm, idxp_hbm):
    mesh = plsc.VectorSubcoreMesh(core_axis_name="c", subcore_axis_name="s")

    @functools.partial(
        pl.kernel,
        mesh=mesh,
        out_type=jax.ShapeDtypeStruct((NUM_CHANNELS * S_PAD,), jnp.float32),
        scratch_types=[
            pltpu.VMEM((TPW,), jnp.int32),
            pltpu.VMEM((CPC * TPW,), jnp.float32),
            pltpu.VMEM_SHARED((NSLOT * NUM_PIXELS,), jnp.float32),
            pltpu.SemaphoreType.DMA,
            pltpu.SemaphoreType.DMA,
            pltpu.SemaphoreType.DMA,
            pltpu.SemaphoreType.DMA,
            pltpu.SemaphoreType.DMA,
        ],
    )
    def k(img_hbm, idx_hbm, out_hbm, idx_v, res_v, rows_s,
          sem_s0, sem_s1, sem_g0, sem_g1, sem_w):
        cid = lax.axis_index("c")
        sid = lax.axis_index("s")
        c0 = cid * CPC
        pltpu.sync_copy(idx_hbm.at[pl.ds(sid * TPW, TPW)], idx_v)

        def stage(k_next, sem):
            slot = lax.rem(k_next, NSLOT)
            src = pl.ds(pl.multiple_of((c0 + k_next) * NUM_PIXELS + sid * SEG, 8),
                        SEG)
            dst = pl.ds(pl.multiple_of(slot * NUM_PIXELS + sid * SEG, 8), SEG)
            pltpu.async_copy(img_hbm.at[src], rows_s.at[dst], sem)

        def wait_stage(sem):
            pltpu.make_async_copy(img_hbm.at[pl.ds(0, SEG)],
                                  rows_s.at[pl.ds(0, SEG)], sem).wait()

        def fire_gather(kk, sem):
            slot = lax.rem(kk, NSLOT)
            src = rows_s.at[pl.ds(pl.multiple_of(slot * NUM_PIXELS, 8),
                                  NUM_PIXELS)]
            dst = res_v.at[pl.ds(kk * TPW, TPW)]
            pltpu.async_copy(src.at[idx_v], dst, sem)

        def wait_gather(sem):
            pltpu.make_async_copy(img_hbm.at[pl.ds(0, TPW)],
                                  res_v.at[pl.ds(0, TPW)], sem).wait()

        def fire_write(kk):
            dst = pl.ds(pl.multiple_of((c0 + kk) * S_PAD + sid * TPW, 8), TPW)
            pltpu.async_copy(res_v.at[pl.ds(kk * TPW, TPW)],
                             out_hbm.at[dst], sem_w)

        stage(0, sem_s0)
        stage(1, sem_s1)

        def chan(kk, carry):
            even = lax.rem(kk, 2) == 0

            @pl.when(even)
            def _():
                wait_stage(sem_s0)

            @pl.when(jnp.logical_not(even))
            def _():
                wait_stage(sem_s1)

            plsc.subcore_barrier()

            @pl.when(even)
            def _():
                fire_gather(kk, sem_g0)

            @pl.when(jnp.logical_not(even))
            def _():
                fire_gather(kk, sem_g1)

            @pl.when(jnp.logical_and(kk >= 1, jnp.logical_not(even)))
            def _():
                wait_gather(sem_g0)

            @pl.when(jnp.logical_and(kk >= 1, even))
            def _():
                wait_gather(sem_g1)

            @pl.when(kk >= 1)
            def _():
                fire_write(kk - 1)

            plsc.subcore_barrier()

            @pl.when(jnp.logical_and(even, kk < CPC - 2))
            def _():
                stage(kk + 2, sem_s0)

            @pl.when(jnp.logical_and(jnp.logical_not(even), kk < CPC - 2))
            def _():
                stage(kk + 2, sem_s1)

            return carry

        lax.fori_loop(0, CPC, chan, 0)

        wait_gather(sem_g1)
        fire_write(CPC - 1)
        pltpu.make_async_copy(img_hbm.at[pl.ds(0, CPC * TPW)],
                              res_v, sem_w).wait()

    return k(img_flat_hbm, idxp_hbm)


def kernel(image_flat, idx):
    idx_i = idx.astype(jnp.int32)
    idxp = jnp.pad(idx_i, (0, S_PAD - NUM_SAMPLES))
    out_flat = _sc_gather(image_flat.reshape(-1), idxp)
    image_subset = out_flat.reshape(NUM_CHANNELS, S_PAD)[:, :NUM_SAMPLES]
    return (image_subset, idx)
